# baseline (speedup 1.0000x reference)
import numpy as np
import jax
import jax.numpy as jnp
from jax.experimental import pallas as pl

_B = 4
_MEM = 8192
_G = 5
_D = 512 * 3 * 3
_K = 50
_TILE = 512


def _conv_shift_mats(bsz, g):
    n = bsz * g * g
    mats = np.zeros((9, n, n), np.float32)
    for ki in range(3):
        for kj in range(3):
            o = ki * 3 + kj
            for b in range(bsz):
                for i in range(g):
                    for j in range(g):
                        si, sj = i + ki - 1, j + kj - 1
                        if 0 <= si < g and 0 <= sj < g:
                            mats[o, b * g * g + i * g + j,
                                 b * g * g + si * g + sj] = 1.0
    return mats


def _pool_mat(bsz):
    starts, ends = [0, 1, 3], [2, 4, 5]
    m = np.zeros((bsz * 9, bsz * 25), np.float32)
    for b in range(bsz):
        for i in range(3):
            for j in range(3):
                w = 1.0 / ((ends[i] - starts[i]) * (ends[j] - starts[j]))
                for si in range(starts[i], ends[i]):
                    for sj in range(starts[j], ends[j]):
                        m[b * 9 + i * 3 + j, b * 25 + si * 5 + sj] = w
    return m


def _sel_mat(bsz):
    m = np.zeros((9, bsz, bsz * 9), np.float32)
    for ij in range(9):
        for b in range(bsz):
            m[ij, b, b * 9 + ij] = 1.0
    return m


def _conv(h, g_ref, w_ref, b_ref):
    acc = None
    for o in range(9):
        sh = jnp.dot(g_ref[o], h, preferred_element_type=jnp.float32)
        t = jnp.dot(sh, w_ref[o], preferred_element_type=jnp.float32)
        acc = t if acc is None else acc + t
    return jax.nn.relu(acc + b_ref[...])


def _prep_body(xr, g5, g3, poolm, selm, cw1, cb1, cw2, cb2, cw3, cb3, cw4,
               cb4, w1q, dw1, db1, dw2, db2, hq_out, dec_out):
    h = _conv(xr[...], g5, cw1, cb1)
    h = _conv(h, g5, cw2, cb2)
    h = _conv(h, g5, cw3, cb3)
    h = jnp.dot(poolm[...], h, preferred_element_type=jnp.float32)
    y4 = _conv(h, g3, cw4, cb4)
    hq = None
    dd = None
    for ij in range(9):
        rows = jnp.dot(selm[ij], y4, preferred_element_type=jnp.float32)
        tq = jnp.dot(rows, w1q[ij], preferred_element_type=jnp.float32)
        td = jnp.dot(rows, dw1[ij], preferred_element_type=jnp.float32)
        hq = tq if hq is None else hq + tq
        dd = td if dd is None else dd + td
    d1 = jax.nn.relu(dd + db1[...])
    dec = jnp.dot(d1, dw2[...], preferred_element_type=jnp.float32) + db2[...]
    hq_out[...] = hq
    dec_out[...] = dec


def _sims_body(mem, w1k, hq, b1, w2, b2, w3, out):
    hk = jnp.dot(mem[...], w1k[...], preferred_element_type=jnp.float32)
    for b in range(_B):
        h1 = jax.nn.relu(hk + hq[b:b + 1, :] + b1[...])
        h2 = jax.nn.relu(
            jnp.dot(h1, w2[...], preferred_element_type=jnp.float32) + b2[...])
        out[0, b, :] = jnp.sum(h2 * w3[...], axis=1)


def _final_body(sims, mo, dec, out):
    s = sims[...]
    m = jnp.max(s, axis=1, keepdims=True)

    def knock(_, v):
        mm = jnp.max(v, axis=1, keepdims=True)
        return jnp.where(v >= mm, -1e30, v)

    v = jax.lax.fori_loop(0, _K - 1, knock, s)
    t = jnp.max(v, axis=1, keepdims=True)
    w = jnp.where(s >= t, jnp.exp((s - m) * 10.0), 0.0)
    scale = 0.8 / jnp.sum(w, axis=1, keepdims=True)
    moi = mo[...]
    for c in range(10):
        ohc = (moi == c).astype(jnp.float32)
        omc = jnp.dot(w, ohc, preferred_element_type=jnp.float32)
        out[:, c, :] = omc * scale + 0.2 * dec[:, c, :]


def kernel(x, conv1_w, conv1_b, conv2_w, conv2_b, conv3_w, conv3_b, conv4_w,
           conv4_b, sim_w1, sim_b1, sim_w2, sim_b2, sim_w3, sim_b3, dec_w1,
           dec_b1, dec_w2, dec_b2, memory_inputs, memory_outputs):
    f32 = jnp.float32
    xr = jnp.transpose(x, (0, 2, 3, 1)).reshape(_B * _G * _G, 10)
    g5 = jnp.asarray(_conv_shift_mats(_B, 5))
    g3 = jnp.asarray(_conv_shift_mats(_B, 3))
    poolm = jnp.asarray(_pool_mat(_B))
    selm = jnp.asarray(_sel_mat(_B))

    def cmat(w):
        return jnp.transpose(w, (2, 3, 1, 0)).reshape(9, w.shape[1], w.shape[0])

    def fold(w):
        return jnp.transpose(w.reshape(512, 9, w.shape[1]), (1, 0, 2))

    hq, dec = pl.pallas_call(
        _prep_body,
        out_shape=(jax.ShapeDtypeStruct((_B, 512), f32),
                   jax.ShapeDtypeStruct((_B, 250), f32)),
    )(xr, g5, g3, poolm, selm,
      cmat(conv1_w), conv1_b.reshape(1, -1),
      cmat(conv2_w), conv2_b.reshape(1, -1),
      cmat(conv3_w), conv3_b.reshape(1, -1),
      cmat(conv4_w), conv4_b.reshape(1, -1),
      fold(sim_w1[:_D]), fold(dec_w1), dec_b1.reshape(1, -1),
      dec_w2, dec_b2.reshape(1, -1))

    nblk = _MEM // _TILE
    sims3 = pl.pallas_call(
        _sims_body,
        grid=(nblk,),
        in_specs=[
            pl.BlockSpec((_TILE, _D), lambda i: (i, 0)),
            pl.BlockSpec((_D, 512), lambda i: (0, 0)),
            pl.BlockSpec((_B, 512), lambda i: (0, 0)),
            pl.BlockSpec((1, 512), lambda i: (0, 0)),
            pl.BlockSpec((512, 256), lambda i: (0, 0)),
            pl.BlockSpec((1, 256), lambda i: (0, 0)),
            pl.BlockSpec((1, 256), lambda i: (0, 0)),
        ],
        out_specs=pl.BlockSpec((1, _B, _TILE), lambda i: (i, 0, 0)),
        out_shape=jax.ShapeDtypeStruct((nblk, _B, _TILE), f32),
    )(memory_inputs, sim_w1[_D:], hq, sim_b1.reshape(1, -1), sim_w2,
      sim_b2.reshape(1, -1), sim_w3.reshape(1, -1))

    sims = jnp.transpose(sims3, (1, 0, 2)).reshape(_B, _MEM)
    out = pl.pallas_call(
        _final_body,
        out_shape=jax.ShapeDtypeStruct((_B, 10, 25), f32),
    )(sims, memory_outputs.reshape(_MEM, _G * _G), dec.reshape(_B, 10, 25))
    return out.reshape(_B, 10, _G, _G)


# speedup vs baseline: 1.2578x; 1.2578x over previous
import numpy as np
import jax
import jax.numpy as jnp
from jax.experimental import pallas as pl

_B = 4
_MEM = 8192
_G = 5
_D = 512 * 3 * 3
_K = 50
_TILE = 512


def _conv_shift_mats(bsz, g):
    n = bsz * g * g
    mats = np.zeros((9, n, n), np.float32)
    for ki in range(3):
        for kj in range(3):
            o = ki * 3 + kj
            for b in range(bsz):
                for i in range(g):
                    for j in range(g):
                        si, sj = i + ki - 1, j + kj - 1
                        if 0 <= si < g and 0 <= sj < g:
                            mats[o, b * g * g + i * g + j,
                                 b * g * g + si * g + sj] = 1.0
    return mats


def _pool_mat(bsz):
    starts, ends = [0, 1, 3], [2, 4, 5]
    m = np.zeros((bsz * 9, bsz * 25), np.float32)
    for b in range(bsz):
        for i in range(3):
            for j in range(3):
                w = 1.0 / ((ends[i] - starts[i]) * (ends[j] - starts[j]))
                for si in range(starts[i], ends[i]):
                    for sj in range(starts[j], ends[j]):
                        m[b * 9 + i * 3 + j, b * 25 + si * 5 + sj] = w
    return m


def _bf(a):
    return a.astype(jnp.bfloat16)


def _bdot(a, b):
    return jnp.dot(_bf(a), _bf(b), preferred_element_type=jnp.float32)


def _xdot(a, b):
    return jnp.dot(a, b, preferred_element_type=jnp.float32,
                   precision=jax.lax.Precision.HIGHEST)


def _conv(h, g_ref, w_ref, b_ref):
    acc = None
    for o in range(9):
        sh = _xdot(g_ref[o], h)
        t = _bdot(sh, w_ref[o])
        acc = t if acc is None else acc + t
    return jax.nn.relu(acc + b_ref[...])


def _enc_body(xr, g5, g3, poolm, cw1, cb1, cw2, cb2, cw3, cb3, cw4, cb4,
              y4_out):
    h = _conv(xr[...], g5, cw1, cb1)
    h = _conv(h, g5, cw2, cb2)
    h = _conv(h, g5, cw3, cb3)
    h = _xdot(poolm[...], h)
    y4_out[...] = _conv(h, g3, cw4, cb4)


def _head_body(enc_ref, dw1, db1, dw2, db2, dec_out):
    enc = enc_ref[...]
    dd = _bdot(enc, dw1[...])
    d1 = jax.nn.relu(dd + db1[...])
    dec_out[...] = _bdot(d1, dw2[...]) + db2[...]


def _sims_body(mem, w1k, hq, b1, w2, b2, w3, out):
    hk = _bdot(mem[...], w1k[...])
    w3r = _bf(w3[...]).astype(jnp.float32)
    for b in range(_B):
        h1 = jax.nn.relu(hk + hq[b:b + 1, :] + b1[...])
        h2 = jax.nn.relu(_bdot(h1, w2[...]) + b2[...])
        out[0, b, :] = jnp.sum(_bf(h2).astype(jnp.float32) * w3r, axis=1)


def _final_body(sims, mo, dec, out):
    s = sims[...]
    m = jnp.max(s, axis=1, keepdims=True)

    def knock(_, v):
        mm = jnp.max(v, axis=1, keepdims=True)
        return jnp.where(v >= mm, -1e30, v)

    v = jax.lax.fori_loop(0, _K - 1, knock, s)
    t = jnp.max(v, axis=1, keepdims=True)
    w = jnp.where(s >= t, jnp.exp((s - m) * 10.0), 0.0)
    scale = 0.8 / jnp.sum(w, axis=1, keepdims=True)
    moi = mo[...]
    for c in range(10):
        ohc = (moi == c).astype(jnp.float32)
        omc = _xdot(w, ohc)
        out[:, c, :] = omc * scale + 0.2 * dec[:, c, :]


def kernel(x, conv1_w, conv1_b, conv2_w, conv2_b, conv3_w, conv3_b, conv4_w,
           conv4_b, sim_w1, sim_b1, sim_w2, sim_b2, sim_w3, sim_b3, dec_w1,
           dec_b1, dec_w2, dec_b2, memory_inputs, memory_outputs):
    f32 = jnp.float32
    xr = jnp.transpose(x, (0, 2, 3, 1)).reshape(_B * _G * _G, 10)
    g5 = jnp.asarray(_conv_shift_mats(_B, 5))
    g3 = jnp.asarray(_conv_shift_mats(_B, 3))
    poolm = jnp.asarray(_pool_mat(_B))

    def cmat(w):
        return jnp.transpose(w, (2, 3, 1, 0)).reshape(9, w.shape[1], w.shape[0])

    y4 = pl.pallas_call(
        _enc_body,
        out_shape=jax.ShapeDtypeStruct((_B * 9, 512), f32),
    )(xr, g5, g3, poolm,
      cmat(conv1_w), conv1_b.reshape(1, -1),
      cmat(conv2_w), conv2_b.reshape(1, -1),
      cmat(conv3_w), conv3_b.reshape(1, -1),
      cmat(conv4_w), conv4_b.reshape(1, -1))
    enc = jnp.transpose(y4.reshape(_B, 9, 512), (0, 2, 1)).reshape(_B, _D)

    hq = jnp.dot(enc, sim_w1[:_D])

    dec = pl.pallas_call(
        _head_body,
        out_shape=jax.ShapeDtypeStruct((_B, 250), f32),
    )(enc, dec_w1, dec_b1.reshape(1, -1), dec_w2, dec_b2.reshape(1, -1))

    nblk = _MEM // _TILE
    sims3 = pl.pallas_call(
        _sims_body,
        grid=(nblk,),
        in_specs=[
            pl.BlockSpec((_TILE, _D), lambda i: (i, 0)),
            pl.BlockSpec((_D, 512), lambda i: (1, 0)),
            pl.BlockSpec((_B, 512), lambda i: (0, 0)),
            pl.BlockSpec((1, 512), lambda i: (0, 0)),
            pl.BlockSpec((512, 256), lambda i: (0, 0)),
            pl.BlockSpec((1, 256), lambda i: (0, 0)),
            pl.BlockSpec((1, 256), lambda i: (0, 0)),
        ],
        out_specs=pl.BlockSpec((1, _B, _TILE), lambda i: (i, 0, 0)),
        out_shape=jax.ShapeDtypeStruct((nblk, _B, _TILE), f32),
    )(memory_inputs, sim_w1, hq, sim_b1.reshape(1, -1), sim_w2,
      sim_b2.reshape(1, -1), sim_w3.reshape(1, -1))

    sims = jnp.transpose(sims3, (1, 0, 2)).reshape(_B, _MEM)
    out = pl.pallas_call(
        _final_body,
        out_shape=jax.ShapeDtypeStruct((_B, 10, 25), f32),
    )(sims, memory_outputs.reshape(_MEM, _G * _G), dec.reshape(_B, 10, 25))
    return out.reshape(_B, 10, _G, _G)
